# baseline (speedup 1.0000x reference)
import functools

import jax
import jax.numpy as jnp
from jax import lax
from jax.experimental import pallas as pl
from jax.experimental.pallas import tpu as pltpu
from jax.experimental.pallas import tpu_sc as plsc

_B, _S, _D = 1024, 200, 128
_E = 32
_N = _B * _S


def _sc_concat_embed(x2d, ner1d, table):
    info = plsc.get_sparse_core_info()
    nw = info.num_cores * info.num_subcores
    n_per_w = _N // nw
    chunk = 1600
    steps = n_per_w // chunk

    mesh = plsc.VectorSubcoreMesh(core_axis_name="c", subcore_axis_name="s")

    @functools.partial(
        pl.kernel,
        mesh=mesh,
        out_type=jax.ShapeDtypeStruct((_N, _D + _E), jnp.float32),
        compiler_params=pltpu.CompilerParams(use_tc_tiling_on_sc=False),
        scratch_types=[
            pltpu.VMEM((chunk,), jnp.int32),
            pltpu.VMEM((chunk,), jnp.int32),
            pltpu.VMEM((chunk, _E), jnp.float32),
            pltpu.VMEM((chunk, _E), jnp.float32),
            pltpu.SemaphoreType.DMA,
            pltpu.SemaphoreType.DMA,
            pltpu.SemaphoreType.DMA,
            pltpu.SemaphoreType.DMA,
            pltpu.SemaphoreType.DMA,
            pltpu.SemaphoreType.DMA,
            pltpu.SemaphoreType.DMA,
        ],
    )
    def k(x_hbm, ner_hbm, table_hbm, out_hbm,
          idx0, idx1, rows0, rows1,
          sem_x, sem_i0, sem_i1, sem_g0, sem_g1, sem_s0, sem_s1):
        wid = lax.axis_index("s") * info.num_cores + lax.axis_index("c")
        base = wid * n_per_w
        idx = (idx0, idx1)
        rows = (rows0, rows1)
        sem_i = (sem_i0, sem_i1)
        sem_g = (sem_g0, sem_g1)
        sem_s = (sem_s0, sem_s1)

        xcopy = pltpu.async_copy(
            x_hbm.at[pl.ds(base, n_per_w), :],
            out_hbm.at[pl.ds(base, n_per_w), pl.ds(0, _D)],
            sem_x,
        )

        def off(s):
            return base + s * chunk

        def load(s):
            b = s % 2
            return pltpu.async_copy(ner_hbm.at[pl.ds(off(s), chunk)], idx[b],
                                    sem_i[b])

        def gather(s):
            b = s % 2
            return pltpu.async_copy(table_hbm.at[idx[b]], rows[b], sem_g[b])

        def store(s):
            b = s % 2
            return pltpu.async_copy(
                rows[b], out_hbm.at[pl.ds(off(s), chunk), pl.ds(_D, _E)],
                sem_s[b])

        l0 = load(0)
        l1 = load(1)
        l0.wait()
        g0 = gather(0)
        l1.wait()
        g1 = gather(1)
        g0.wait()
        s0 = store(0)
        l2 = load(2)
        g1.wait()
        s1 = store(1)
        l3 = load(3)
        s0.wait()
        l2.wait()
        g2 = gather(2)
        s1.wait()
        l3.wait()
        g3 = gather(3)
        g2.wait()
        s2 = store(2)
        g3.wait()
        s3 = store(3)
        s2.wait()
        s3.wait()
        xcopy.wait()

    return k(x2d, ner1d, table)


def kernel(x, ner, pos, entity_table):
    del pos
    x2d = x.reshape(_N, _D)
    ner1d = ner.reshape(_N).astype(jnp.int32)
    out = _sc_concat_embed(x2d, ner1d, entity_table)
    return out.reshape(_B, _S, _D + _E)


# speedup vs baseline: 7.4679x; 7.4679x over previous
import functools

import jax
import jax.numpy as jnp
from jax import lax
from jax.experimental import pallas as pl
from jax.experimental.pallas import tpu as pltpu
from jax.experimental.pallas import tpu_sc as plsc

_B, _S, _D = 1024, 200, 128
_E = 32
_N = _B * _S

_CX = 200
_CG = 640


def _sc_concat_embed(x2d, ner1d, table):
    info = plsc.get_sparse_core_info()
    nw = info.num_cores * info.num_subcores
    n_per_w = _N // nw
    steps_x = n_per_w // _CX
    steps_g = n_per_w // _CG

    mesh = plsc.VectorSubcoreMesh(core_axis_name="c", subcore_axis_name="s")

    @functools.partial(
        pl.kernel,
        mesh=mesh,
        out_type=jax.ShapeDtypeStruct((_N, _D + _E), jnp.float32),
        compiler_params=pltpu.CompilerParams(use_tc_tiling_on_sc=False),
        scratch_types=[
            pltpu.VMEM((3, _CX, _D), jnp.float32),
            pltpu.VMEM((2, _CG), jnp.int32),
            pltpu.VMEM((2, _CG, _E), jnp.float32),
            pltpu.SemaphoreType.DMA((3,)),
            pltpu.SemaphoreType.DMA((3,)),
            pltpu.SemaphoreType.DMA((2,)),
            pltpu.SemaphoreType.DMA((2,)),
            pltpu.SemaphoreType.DMA((2,)),
        ],
    )
    def k(x_hbm, ner_hbm, table_hbm, out_hbm,
          xbuf, idx, rows, sem_xl, sem_xs, sem_i, sem_g, sem_r):
        wid = lax.axis_index("s") * info.num_cores + lax.axis_index("c")
        base = wid * n_per_w

        def xload(t):
            b = t % 3
            return pltpu.async_copy(
                x_hbm.at[pl.ds(base + t * _CX, _CX), :], xbuf.at[b],
                sem_xl.at[b])

        def xstore(t):
            b = t % 3
            return pltpu.async_copy(
                xbuf.at[b],
                out_hbm.at[pl.ds(base + t * _CX, _CX), pl.ds(0, _D)],
                sem_xs.at[b])

        def iload(s):
            b = s % 2
            return pltpu.async_copy(
                ner_hbm.at[pl.ds(base + s * _CG, _CG)], idx.at[b],
                sem_i.at[b])

        def gath(s):
            b = s % 2
            return pltpu.async_copy(table_hbm.at[idx.at[b]], rows.at[b],
                                    sem_g.at[b])

        def rstore(s):
            b = s % 2
            return pltpu.async_copy(
                rows.at[b],
                out_hbm.at[pl.ds(base + s * _CG, _CG), pl.ds(_D, _E)],
                sem_r.at[b])

        xl = [None] * 3
        xs = [None] * 3
        gi = [None] * 2
        gg = [None] * 2
        gr = [None] * 2

        def xstep(t):
            b = t % 3
            if t >= 1 and t + 2 < steps_x:
                xs[(t - 1) % 3].wait()
                xl[(t - 1) % 3] = xload(t + 2)
            xl[b].wait()
            xs[b] = xstore(t)

        def gstep(s):
            b = s % 2
            if s >= 1:
                gg[1 - b].wait()
                gr[1 - b] = rstore(s - 1)
                if s + 1 < steps_g:
                    gi[1 - b] = iload(s + 1)
            if s >= 2:
                gr[b].wait()
            gi[b].wait()
            gg[b] = gath(s)

        gi[0] = iload(0)
        gi[1] = iload(1)
        xl[0] = xload(0)
        xl[1] = xload(1)
        xl[2] = xload(2)

        ratio = steps_x // steps_g
        g_issued = 0
        for t in range(steps_x):
            if t % ratio == 0 and g_issued < steps_g:
                gstep(g_issued)
                g_issued += 1
            xstep(t)
        while g_issued < steps_g:
            gstep(g_issued)
            g_issued += 1

        gg[(steps_g - 1) % 2].wait()
        gr[(steps_g - 1) % 2] = rstore(steps_g - 1)
        gr[steps_g % 2].wait()
        gr[(steps_g - 1) % 2].wait()
        xs[(steps_x - 3) % 3].wait()
        xs[(steps_x - 2) % 3].wait()
        xs[(steps_x - 1) % 3].wait()

    return k(x2d, ner1d, table)


def kernel(x, ner, pos, entity_table):
    del pos
    x2d = x.reshape(_N, _D)
    ner1d = ner.reshape(_N).astype(jnp.int32)
    out = _sc_concat_embed(x2d, ner1d, entity_table)
    return out.reshape(_B, _S, _D + _E)


# speedup vs baseline: 8.0711x; 1.0808x over previous
import functools

import jax
import jax.numpy as jnp
from jax import lax
from jax.experimental import pallas as pl
from jax.experimental.pallas import tpu as pltpu
from jax.experimental.pallas import tpu_sc as plsc

_B, _S, _D = 1024, 200, 128
_E = 32
_N = _B * _S

_CG = 640
_CX = 160
_CE = 160


def _ring3(steps, load, store):
    ld = [None] * 3
    st = [None] * 3

    def prime():
        for b in range(3):
            ld[b] = load(b)

    def step(t):
        b = t % 3
        if t >= 1 and t + 2 < steps:
            st[(t - 1) % 3].wait()
            ld[(t - 1) % 3] = load(t + 2)
        ld[b].wait()
        st[b] = store(t)

    def drain():
        for t in range(max(steps - 3, 0), steps):
            st[t % 3].wait()

    return prime, step, drain


def _sc_gather(ner1d, table):
    info = plsc.get_sparse_core_info()
    nw = info.num_cores * info.num_subcores
    n_per_w = _N // nw
    steps = n_per_w // _CG

    mesh = plsc.VectorSubcoreMesh(core_axis_name="c", subcore_axis_name="s")

    @functools.partial(
        pl.kernel,
        mesh=mesh,
        out_type=jax.ShapeDtypeStruct((_N, _E), jnp.float32),
        compiler_params=pltpu.CompilerParams(use_tc_tiling_on_sc=False),
        scratch_types=[
            pltpu.VMEM((2, _CG), jnp.int32),
            pltpu.VMEM((2, _CG, _E), jnp.float32),
            pltpu.SemaphoreType.DMA((2,)),
            pltpu.SemaphoreType.DMA((2,)),
            pltpu.SemaphoreType.DMA((2,)),
        ],
    )
    def k(ner_hbm, table_hbm, emb_hbm, idx, rows, sem_i, sem_g, sem_r):
        wid = lax.axis_index("s") * info.num_cores + lax.axis_index("c")
        base = wid * n_per_w

        def iload(s):
            b = s % 2
            return pltpu.async_copy(
                ner_hbm.at[pl.ds(base + s * _CG, _CG)], idx.at[b],
                sem_i.at[b])

        def gath(s):
            b = s % 2
            return pltpu.async_copy(table_hbm.at[idx.at[b]], rows.at[b],
                                    sem_g.at[b])

        def rstore(s):
            b = s % 2
            return pltpu.async_copy(
                rows.at[b],
                emb_hbm.at[pl.ds(base + s * _CG, _CG), :],
                sem_r.at[b])

        gi = [None] * 2
        gg = [None] * 2
        gr = [None] * 2

        gi[0] = iload(0)
        gi[1] = iload(1)
        for s in range(steps):
            b = s % 2
            if s >= 1:
                gg[1 - b].wait()
                gr[1 - b] = rstore(s - 1)
                if s + 1 < steps:
                    gi[1 - b] = iload(s + 1)
            if s >= 2:
                gr[b].wait()
            gi[b].wait()
            gg[b] = gath(s)
        gg[(steps - 1) % 2].wait()
        gr[(steps - 1) % 2] = rstore(steps - 1)
        gr[steps % 2].wait()
        gr[(steps - 1) % 2].wait()

    return k(ner1d, table)


def _sc_concat(x2d, emb4):
    info = plsc.get_sparse_core_info()
    nw = info.num_cores * info.num_subcores
    n_per_w = _N // nw
    steps_x = n_per_w // _CX
    steps_e = n_per_w // _CE

    mesh = plsc.VectorSubcoreMesh(core_axis_name="c", subcore_axis_name="s")

    @functools.partial(
        pl.kernel,
        mesh=mesh,
        out_type=jax.ShapeDtypeStruct((_N, _D + _E), jnp.float32),
        scratch_types=[
            pltpu.VMEM((3, _CX, _D), jnp.float32),
            pltpu.VMEM((3, _CE, _E), jnp.float32),
            pltpu.SemaphoreType.DMA((3,)),
            pltpu.SemaphoreType.DMA((3,)),
            pltpu.SemaphoreType.DMA((3,)),
            pltpu.SemaphoreType.DMA((3,)),
        ],
    )
    def k(x_hbm, emb_hbm, out_hbm, xbuf, ebuf,
          sem_xl, sem_xs, sem_el, sem_es):
        wid = lax.axis_index("s") * info.num_cores + lax.axis_index("c")
        base = wid * n_per_w

        xp, xstep, xdrain = _ring3(
            steps_x,
            lambda t: pltpu.async_copy(
                x_hbm.at[pl.ds(base + t * _CX, _CX), :], xbuf.at[t % 3],
                sem_xl.at[t % 3]),
            lambda t: pltpu.async_copy(
                xbuf.at[t % 3],
                out_hbm.at[pl.ds(base + t * _CX, _CX), pl.ds(0, _D)],
                sem_xs.at[t % 3]),
        )
        ep, estep, edrain = _ring3(
            steps_e,
            lambda s: pltpu.async_copy(
                emb_hbm.at[pl.ds(base + s * _CE, _CE), :],
                ebuf.at[s % 3], sem_el.at[s % 3]),
            lambda s: pltpu.async_copy(
                ebuf.at[s % 3],
                out_hbm.at[pl.ds(base + s * _CE, _CE), pl.ds(_D, _E)],
                sem_es.at[s % 3]),
        )

        xp()
        ep()
        e_issued = 0
        for t in range(steps_x):
            while e_issued * steps_x < (t + 1) * steps_e:
                estep(e_issued)
                e_issued += 1
            xstep(t)
        while e_issued < steps_e:
            estep(e_issued)
            e_issued += 1
        xdrain()
        edrain()

    return k(x2d, emb4)


def kernel(x, ner, pos, entity_table):
    del pos
    x2d = x.reshape(_N, _D)
    ner1d = ner.reshape(_N).astype(jnp.int32)
    emb = _sc_gather(ner1d, entity_table)
    out = _sc_concat(x2d, emb)
    return out.reshape(_B, _S, _D + _E)
